# baseline (speedup 1.0000x reference)
import jax
import jax.numpy as jnp
from jax.experimental import pallas as pl
from jax.experimental.pallas import tpu as pltpu

N = 10000
F = 128
R2 = N // 2
BLOCK = 2504


def _body(x_ref, w1_ref, b1_ref, w2_ref, b2_ref, o_ref):
    h = jnp.dot(x_ref[...], w1_ref[...], preferred_element_type=jnp.float32)
    h = jnp.maximum(h + b1_ref[...], 0.0)
    out = jnp.dot(h, w2_ref[...], preferred_element_type=jnp.float32)
    out = out + b2_ref[...]
    for k in (0, 1):
        o = out[:, k * F:(k + 1) * F]
        m = jnp.max(o, axis=-1, keepdims=True)
        l = o - m
        o_ref[:, k * F:(k + 1) * F] = l - jnp.log(
            jnp.sum(jnp.exp(l), axis=-1, keepdims=True))


def _blockdiag(w):
    z = jnp.zeros((F, F), w.dtype)
    return jnp.block([[w, z], [z, w]])


def kernel(x, W1, b1, W2, b2, edge_index):
    del edge_index
    w1big = _blockdiag(W1.T)
    w2big = _blockdiag(W2.T)
    b1big = jnp.concatenate([b1, b1]).reshape(1, 2 * F)
    b2big = jnp.concatenate([b2, b2]).reshape(1, 2 * F)
    x2 = x.reshape(R2, 2 * F)
    grid = (pl.cdiv(R2, BLOCK),)
    y2 = pl.pallas_call(
        _body,
        grid=grid,
        in_specs=[
            pl.BlockSpec((BLOCK, 2 * F), lambda i: (i, 0)),
            pl.BlockSpec((2 * F, 2 * F), lambda i: (0, 0)),
            pl.BlockSpec((1, 2 * F), lambda i: (0, 0)),
            pl.BlockSpec((2 * F, 2 * F), lambda i: (0, 0)),
            pl.BlockSpec((1, 2 * F), lambda i: (0, 0)),
        ],
        out_specs=pl.BlockSpec((BLOCK, 2 * F), lambda i: (i, 0)),
        out_shape=jax.ShapeDtypeStruct((R2, 2 * F), jnp.float32),
        compiler_params=pltpu.CompilerParams(
            dimension_semantics=("arbitrary",),
        ),
    )(x2, w1big, b1big, w2big, b2big)
    return y2.reshape(N, F)


# speedup vs baseline: 2.0774x; 2.0774x over previous
import jax
import jax.numpy as jnp
from jax.experimental import pallas as pl
from jax.experimental.pallas import tpu as pltpu

N = 10000
F = 128
H = N // 2
BLOCK = 2504


def _body(x_ref, w1_ref, b1_ref, w2_ref, b2_ref, o_ref):
    x2 = jnp.concatenate([x_ref[0], x_ref[1]], axis=1)
    h = jnp.dot(x2, w1_ref[...], preferred_element_type=jnp.float32)
    h = jnp.maximum(h + b1_ref[...], 0.0)
    out = jnp.dot(h, w2_ref[...], preferred_element_type=jnp.float32)
    out = out + b2_ref[...]
    for k in (0, 1):
        o = out[:, k * F:(k + 1) * F]
        m = jnp.max(o, axis=-1, keepdims=True)
        l = o - m
        o_ref[k] = l - jnp.log(jnp.sum(jnp.exp(l), axis=-1, keepdims=True))


def _blockdiag(w):
    z = jnp.zeros((F, F), w.dtype)
    return jnp.block([[w, z], [z, w]])


def kernel(x, W1, b1, W2, b2, edge_index):
    del edge_index
    w1big = _blockdiag(W1.T)
    w2big = _blockdiag(W2.T)
    b1big = jnp.concatenate([b1, b1]).reshape(1, 2 * F)
    b2big = jnp.concatenate([b2, b2]).reshape(1, 2 * F)
    x3 = x.reshape(2, H, F)
    grid = (pl.cdiv(H, BLOCK),)
    y3 = pl.pallas_call(
        _body,
        grid=grid,
        in_specs=[
            pl.BlockSpec((2, BLOCK, F), lambda i: (0, i, 0)),
            pl.BlockSpec((2 * F, 2 * F), lambda i: (0, 0)),
            pl.BlockSpec((1, 2 * F), lambda i: (0, 0)),
            pl.BlockSpec((2 * F, 2 * F), lambda i: (0, 0)),
            pl.BlockSpec((1, 2 * F), lambda i: (0, 0)),
        ],
        out_specs=pl.BlockSpec((2, BLOCK, F), lambda i: (0, i, 0)),
        out_shape=jax.ShapeDtypeStruct((2, H, F), jnp.float32),
        compiler_params=pltpu.CompilerParams(
            dimension_semantics=("arbitrary",),
        ),
    )(x3, w1big, b1big, w2big, b2big)
    return y3.reshape(N, F)


# speedup vs baseline: 2.5345x; 1.2201x over previous
import jax
import jax.numpy as jnp
from jax.experimental import pallas as pl
from jax.experimental.pallas import tpu as pltpu

N = 10000
F = 128
BLOCK = 5000
SUB = 1000


def _body(x_ref, w1_ref, b1_ref, w2_ref, b2_ref, o_ref):
    for s in range(BLOCK // SUB):
        xs = x_ref[pl.ds(s * SUB, SUB), :]
        h = jnp.dot(xs, w1_ref[...], preferred_element_type=jnp.float32)
        h = jnp.maximum(h + b1_ref[...], 0.0)
        out = jnp.dot(h, w2_ref[...], preferred_element_type=jnp.float32)
        out = out + b2_ref[...]
        m = jnp.max(out, axis=-1, keepdims=True)
        l = out - m
        o_ref[pl.ds(s * SUB, SUB), :] = l - jnp.log(
            jnp.sum(jnp.exp(l), axis=-1, keepdims=True))


def kernel(x, W1, b1, W2, b2, edge_index):
    del edge_index
    w1t = W1.T
    w2t = W2.T
    b1r = b1.reshape(1, F)
    b2r = b2.reshape(1, F)
    grid = (N // BLOCK,)
    return pl.pallas_call(
        _body,
        grid=grid,
        in_specs=[
            pl.BlockSpec((BLOCK, F), lambda i: (i, 0)),
            pl.BlockSpec((F, F), lambda i: (0, 0)),
            pl.BlockSpec((1, F), lambda i: (0, 0)),
            pl.BlockSpec((F, F), lambda i: (0, 0)),
            pl.BlockSpec((1, F), lambda i: (0, 0)),
        ],
        out_specs=pl.BlockSpec((BLOCK, F), lambda i: (i, 0)),
        out_shape=jax.ShapeDtypeStruct((N, F), jnp.float32),
        compiler_params=pltpu.CompilerParams(
            dimension_semantics=("arbitrary",),
        ),
    )(x, w1t, b1r, w2t, b2r)
